# Initial kernel scaffold; baseline (speedup 1.0000x reference)
import functools

import jax
import jax.numpy as jnp
from jax import lax
from jax.experimental import pallas as pl
from jax.experimental.pallas import tpu as pltpu
from jax.experimental.pallas import tpu_sc as plsc

N_NODES = 10000
D = 128
N_PAD = 10240
DUMMY = N_NODES
NC = 2
NS = 16
L = 16
NW = NC * NS
CHUNK = 128
ROW_BLK = 1024


def _sc_hist(dsts):
    ept_l = dsts.shape[1]
    mesh = plsc.VectorSubcoreMesh(core_axis_name="c", subcore_axis_name="s")

    @functools.partial(
        pl.kernel, mesh=mesh,
        out_type=jax.ShapeDtypeStruct((NW, N_PAD), jnp.float32),
        scratch_types=[
            pltpu.VMEM((ept_l, L), jnp.int32),
            pltpu.VMEM((N_PAD,), jnp.float32),
        ],
    )
    def k(dst_hbm, out_hbm, idx_v, hist_v):
        cid = lax.axis_index("c")
        sid = lax.axis_index("s")
        wid = sid * NC + cid
        zeros16 = jnp.zeros((L,), jnp.float32)
        ones16 = jnp.ones((L,), jnp.float32)

        @pl.loop(0, N_PAD // L)
        def _(i):
            hist_v[pl.ds(i * L, L)] = zeros16

        pltpu.sync_copy(dst_hbm.at[wid], idx_v)

        @pl.loop(0, ept_l)
        def _(j):
            plsc.addupdate_scatter(hist_v, [idx_v[j]], ones16)

        pltpu.sync_copy(hist_v, out_hbm.at[wid])

    return k(dsts)


def _sc_agg(g, srcs, dsts):
    nchunk = srcs.shape[1]
    rows_per_tile = N_PAD // NS
    mesh = plsc.VectorSubcoreMesh(core_axis_name="c", subcore_axis_name="s")

    @functools.partial(
        pl.kernel, mesh=mesh,
        out_type=jax.ShapeDtypeStruct((NC, N_PAD, D), jnp.float32),
        scratch_types=[
            pltpu.VMEM((nchunk, CHUNK), jnp.int32),
            pltpu.VMEM((nchunk, CHUNK), jnp.int32),
            pltpu.VMEM((CHUNK, D), jnp.float32),
            pltpu.VMEM((CHUNK, D), jnp.float32),
            pltpu.VMEM_SHARED((N_PAD, D), jnp.float32),
            pltpu.SemaphoreType.DMA,
            pltpu.SemaphoreType.DMA,
        ],
    )
    def k(g_hbm, src_hbm, dst_hbm, out_hbm,
          src_v, dst_v, buf_a, buf_b, acc, sem_a, sem_b):
        cid = lax.axis_index("c")
        sid = lax.axis_index("s")
        wid = sid * NC + cid
        zeros16 = jnp.zeros((L,), jnp.float32)

        @pl.loop(0, CHUNK)
        def _(r):
            @pl.loop(0, D // L)
            def _(c):
                buf_a[r, pl.ds(c * L, L)] = zeros16

        @pl.loop(0, rows_per_tile // CHUNK)
        def _(t):
            pltpu.sync_copy(
                buf_a, acc.at[pl.ds(sid * rows_per_tile + t * CHUNK, CHUNK)])

        pltpu.sync_copy(src_hbm.at[wid], src_v)
        pltpu.sync_copy(dst_hbm.at[wid], dst_v)
        plsc.subcore_barrier()

        pltpu.async_copy(g_hbm.at[src_v.at[0]], buf_a, sem_a)

        @pl.loop(0, nchunk, step=2)
        def _(j):
            pltpu.async_copy(g_hbm.at[src_v.at[j + 1]], buf_b, sem_b)
            pltpu.make_async_copy(g_hbm.at[src_v.at[j]], buf_a, sem_a).wait()
            pltpu.sync_copy(buf_a, acc.at[dst_v.at[j]], add=True)

            @pl.when(j + 2 < nchunk)
            def _():
                pltpu.async_copy(g_hbm.at[src_v.at[j + 2]], buf_a, sem_a)

            pltpu.make_async_copy(g_hbm.at[src_v.at[j + 1]], buf_b, sem_b).wait()
            pltpu.sync_copy(buf_b, acc.at[dst_v.at[j + 1]], add=True)

        plsc.subcore_barrier()
        pltpu.sync_copy(
            acc.at[pl.ds(sid * rows_per_tile, rows_per_tile)],
            out_hbm.at[cid].at[pl.ds(sid * rows_per_tile, rows_per_tile)])

    return k(g, srcs, dsts)


def _tc_scale_in(hist, x, w1):
    def body(hist_b, x_b, w1_b, dis_b, g1_b):
        deg = jnp.sum(hist_b[...], axis=0) + 1.0
        dis = lax.rsqrt(deg)
        h = jnp.dot(x_b[...], w1_b[...],
                    preferred_element_type=jnp.float32,
                    precision=lax.Precision.HIGHEST)
        dis_b[...] = dis[:, None]
        g1_b[...] = h * dis[:, None]

    grid = (N_PAD // ROW_BLK,)
    return pl.pallas_call(
        body,
        grid=grid,
        in_specs=[
            pl.BlockSpec((NW, ROW_BLK), lambda b: (0, b)),
            pl.BlockSpec((ROW_BLK, D), lambda b: (b, 0)),
            pl.BlockSpec((D, D), lambda b: (0, 0)),
        ],
        out_specs=[
            pl.BlockSpec((ROW_BLK, 1), lambda b: (b, 0)),
            pl.BlockSpec((ROW_BLK, D), lambda b: (b, 0)),
        ],
        out_shape=[
            jax.ShapeDtypeStruct((N_PAD, 1), jnp.float32),
            jax.ShapeDtypeStruct((N_PAD, D), jnp.float32),
        ],
    )(hist, x, w1)


def _tc_mid(p, g1, dis, b1, w2):
    def body(p_b, g1_b, dis_b, b1_b, w2_b, g2_b):
        dis = dis_b[...]
        a = (p_b[0] + p_b[1] + g1_b[...]) * dis + b1_b[...]
        a = jnp.maximum(a, 0.0)
        h2 = jnp.dot(a, w2_b[...],
                     preferred_element_type=jnp.float32,
                     precision=lax.Precision.HIGHEST)
        g2_b[...] = h2 * dis

    grid = (N_PAD // ROW_BLK,)
    return pl.pallas_call(
        body,
        grid=grid,
        in_specs=[
            pl.BlockSpec((NC, ROW_BLK, D), lambda b: (0, b, 0)),
            pl.BlockSpec((ROW_BLK, D), lambda b: (b, 0)),
            pl.BlockSpec((ROW_BLK, 1), lambda b: (b, 0)),
            pl.BlockSpec((1, D), lambda b: (0, 0)),
            pl.BlockSpec((D, D), lambda b: (0, 0)),
        ],
        out_specs=pl.BlockSpec((ROW_BLK, D), lambda b: (b, 0)),
        out_shape=jax.ShapeDtypeStruct((N_PAD, D), jnp.float32),
    )(p, g1, dis, b1, w2)


def _tc_out(q, g2, dis, b2):
    def body(q_b, g2_b, dis_b, b2_b, o_b):
        o_b[...] = (q_b[0] + q_b[1] + g2_b[...]) * dis_b[...] + b2_b[...]

    grid = (N_PAD // ROW_BLK,)
    return pl.pallas_call(
        body,
        grid=grid,
        in_specs=[
            pl.BlockSpec((NC, ROW_BLK, D), lambda b: (0, b, 0)),
            pl.BlockSpec((ROW_BLK, D), lambda b: (b, 0)),
            pl.BlockSpec((ROW_BLK, 1), lambda b: (b, 0)),
            pl.BlockSpec((1, D), lambda b: (0, 0)),
        ],
        out_specs=pl.BlockSpec((ROW_BLK, D), lambda b: (b, 0)),
        out_shape=jax.ShapeDtypeStruct((N_PAD, D), jnp.float32),
    )(q, g2, dis, b2)


def kernel(x, edge_index, W1, b1, W2, b2):
    n_edges = edge_index.shape[1]
    ept = -(-n_edges // NW)
    ept = -(-ept // CHUNK) * CHUNK
    e_pad = ept * NW
    nchunk = ept // CHUNK

    src = edge_index[0].astype(jnp.int32)
    dst = edge_index[1].astype(jnp.int32)
    pad = e_pad - n_edges
    src_p = jnp.concatenate([src, jnp.zeros((pad,), jnp.int32)])
    dst_p = jnp.concatenate([dst, jnp.full((pad,), DUMMY, jnp.int32)])
    srcs = src_p.reshape(NW, nchunk, CHUNK)
    dsts = dst_p.reshape(NW, nchunk, CHUNK)
    dsts_hist = dst_p.reshape(NW, ept // L, L)

    x_pad = jnp.zeros((N_PAD, D), x.dtype).at[:N_NODES].set(x)
    b1r = b1.reshape(1, D)
    b2r = b2.reshape(1, D)

    hist = _sc_hist(dsts_hist)
    dis, g1 = _tc_scale_in(hist, x_pad, W1)
    p = _sc_agg(g1, srcs, dsts)
    g2 = _tc_mid(p, g1, dis, b1r, W2)
    q = _sc_agg(g2, srcs, dsts)
    out = _tc_out(q, g2, dis, b2r)
    return out[:N_NODES]


# speedup vs baseline: 8.9563x; 8.9563x over previous
import functools

import jax
import jax.numpy as jnp
from jax import lax
from jax.experimental import pallas as pl
from jax.experimental.pallas import tpu as pltpu
from jax.experimental.pallas import tpu_sc as plsc

N_NODES = 10000
D = 128
N_PAD = 10240
DUMMY = N_NODES
NC = 2
NS = 16
L = 16
NW = NC * NS
CHUNK = 128
ROW_BLK = 1024

_SC_PARAMS = pltpu.CompilerParams(needs_layout_passes=False)


def _sc_hist(dsts):
    ept_l = dsts.shape[1]
    mesh = plsc.VectorSubcoreMesh(core_axis_name="c", subcore_axis_name="s")

    @functools.partial(
        pl.kernel, mesh=mesh, compiler_params=_SC_PARAMS,
        out_type=jax.ShapeDtypeStruct((NW, N_PAD), jnp.float32),
        scratch_types=[
            pltpu.VMEM((ept_l, L), jnp.int32),
            pltpu.VMEM((N_PAD,), jnp.float32),
        ],
    )
    def k(dst_hbm, out_hbm, idx_v, hist_v):
        cid = lax.axis_index("c")
        sid = lax.axis_index("s")
        wid = sid * NC + cid
        zeros16 = jnp.zeros((L,), jnp.float32)
        ones16 = jnp.ones((L,), jnp.float32)

        @pl.loop(0, N_PAD // L)
        def _(i):
            hist_v[pl.ds(i * L, L)] = zeros16

        pltpu.sync_copy(dst_hbm.at[wid], idx_v)

        @pl.loop(0, ept_l)
        def _(j):
            plsc.addupdate_scatter(hist_v, [idx_v[j]], ones16)

        pltpu.sync_copy(hist_v, out_hbm.at[wid])

    return k(dsts)


def _sc_agg(g, srcs, dsts):
    nchunk = srcs.shape[1]
    stage = nchunk // 2
    rows_per_tile = N_PAD // NS
    mesh = plsc.VectorSubcoreMesh(core_axis_name="c", subcore_axis_name="s")

    @functools.partial(
        pl.kernel, mesh=mesh,
        out_type=jax.ShapeDtypeStruct((NC, N_PAD, D), jnp.float32),
        scratch_types=[
            pltpu.VMEM((stage, CHUNK), jnp.int32),
            pltpu.VMEM((stage, CHUNK), jnp.int32),
            pltpu.VMEM((CHUNK, D), jnp.float32),
            pltpu.VMEM((CHUNK, D), jnp.float32),
            pltpu.VMEM_SHARED((N_PAD, D), jnp.float32),
            pltpu.SemaphoreType.DMA,
            pltpu.SemaphoreType.DMA,
        ],
    )
    def k(g_hbm, src_hbm, dst_hbm, out_hbm,
          src_v, dst_v, buf_a, buf_b, acc, sem_a, sem_b):
        cid = lax.axis_index("c")
        sid = lax.axis_index("s")
        wid = sid * NC + cid
        zeros16 = jnp.zeros((L,), jnp.float32)

        @pl.loop(0, CHUNK)
        def _(r):
            @pl.loop(0, D // L)
            def _(c):
                buf_a[r, pl.ds(c * L, L)] = zeros16

        @pl.loop(0, rows_per_tile // CHUNK)
        def _(t):
            pltpu.sync_copy(
                buf_a, acc.at[pl.ds(sid * rows_per_tile + t * CHUNK, CHUNK)])

        plsc.subcore_barrier()

        @pl.loop(0, 2)
        def _(st):
            pltpu.sync_copy(src_hbm.at[wid].at[pl.ds(st * stage, stage)],
                            src_v)
            pltpu.sync_copy(dst_hbm.at[wid].at[pl.ds(st * stage, stage)],
                            dst_v)
            pltpu.async_copy(g_hbm.at[src_v.at[0]], buf_a, sem_a)

            @pl.loop(0, stage, step=2)
            def _(j):
                pltpu.async_copy(g_hbm.at[src_v.at[j + 1]], buf_b, sem_b)
                pltpu.make_async_copy(g_hbm.at[src_v.at[j]], buf_a,
                                      sem_a).wait()
                pltpu.sync_copy(buf_a, acc.at[dst_v.at[j]], add=True)

                @pl.when(j + 2 < stage)
                def _():
                    pltpu.async_copy(g_hbm.at[src_v.at[j + 2]], buf_a, sem_a)

                pltpu.make_async_copy(g_hbm.at[src_v.at[j + 1]], buf_b,
                                      sem_b).wait()
                pltpu.sync_copy(buf_b, acc.at[dst_v.at[j + 1]], add=True)

        plsc.subcore_barrier()
        pltpu.sync_copy(
            acc.at[pl.ds(sid * rows_per_tile, rows_per_tile)],
            out_hbm.at[cid].at[pl.ds(sid * rows_per_tile, rows_per_tile)])

    return k(g, srcs, dsts)


def _tc_scale_in(hist, x, w1):
    def body(hist_b, x_b, w1_b, dis_b, g1_b):
        deg = jnp.sum(hist_b[...], axis=0) + 1.0
        dis = lax.rsqrt(deg)
        h = jnp.dot(x_b[...], w1_b[...],
                    preferred_element_type=jnp.float32,
                    precision=lax.Precision.HIGHEST)
        dis_b[...] = dis[:, None]
        g1_b[...] = h * dis[:, None]

    grid = (N_PAD // ROW_BLK,)
    return pl.pallas_call(
        body,
        grid=grid,
        in_specs=[
            pl.BlockSpec((NW, ROW_BLK), lambda b: (0, b)),
            pl.BlockSpec((ROW_BLK, D), lambda b: (b, 0)),
            pl.BlockSpec((D, D), lambda b: (0, 0)),
        ],
        out_specs=[
            pl.BlockSpec((ROW_BLK, 1), lambda b: (b, 0)),
            pl.BlockSpec((ROW_BLK, D), lambda b: (b, 0)),
        ],
        out_shape=[
            jax.ShapeDtypeStruct((N_PAD, 1), jnp.float32),
            jax.ShapeDtypeStruct((N_PAD, D), jnp.float32),
        ],
    )(hist, x, w1)


def _tc_mid(p, g1, dis, b1, w2):
    def body(p_b, g1_b, dis_b, b1_b, w2_b, g2_b):
        dis = dis_b[...]
        a = (p_b[0] + p_b[1] + g1_b[...]) * dis + b1_b[...]
        a = jnp.maximum(a, 0.0)
        h2 = jnp.dot(a, w2_b[...],
                     preferred_element_type=jnp.float32,
                     precision=lax.Precision.HIGHEST)
        g2_b[...] = h2 * dis

    grid = (N_PAD // ROW_BLK,)
    return pl.pallas_call(
        body,
        grid=grid,
        in_specs=[
            pl.BlockSpec((NC, ROW_BLK, D), lambda b: (0, b, 0)),
            pl.BlockSpec((ROW_BLK, D), lambda b: (b, 0)),
            pl.BlockSpec((ROW_BLK, 1), lambda b: (b, 0)),
            pl.BlockSpec((1, D), lambda b: (0, 0)),
            pl.BlockSpec((D, D), lambda b: (0, 0)),
        ],
        out_specs=pl.BlockSpec((ROW_BLK, D), lambda b: (b, 0)),
        out_shape=jax.ShapeDtypeStruct((N_PAD, D), jnp.float32),
    )(p, g1, dis, b1, w2)


def _tc_out(q, g2, dis, b2):
    def body(q_b, g2_b, dis_b, b2_b, o_b):
        o_b[...] = (q_b[0] + q_b[1] + g2_b[...]) * dis_b[...] + b2_b[...]

    grid = (N_PAD // ROW_BLK,)
    return pl.pallas_call(
        body,
        grid=grid,
        in_specs=[
            pl.BlockSpec((NC, ROW_BLK, D), lambda b: (0, b, 0)),
            pl.BlockSpec((ROW_BLK, D), lambda b: (b, 0)),
            pl.BlockSpec((ROW_BLK, 1), lambda b: (b, 0)),
            pl.BlockSpec((1, D), lambda b: (0, 0)),
        ],
        out_specs=pl.BlockSpec((ROW_BLK, D), lambda b: (b, 0)),
        out_shape=jax.ShapeDtypeStruct((N_PAD, D), jnp.float32),
    )(q, g2, dis, b2)


def kernel(x, edge_index, W1, b1, W2, b2):
    n_edges = edge_index.shape[1]
    ept = -(-n_edges // NW)
    ept = -(-ept // (16 * CHUNK)) * (16 * CHUNK)
    e_pad = ept * NW
    nchunk = ept // CHUNK

    src = edge_index[0].astype(jnp.int32)
    dst = edge_index[1].astype(jnp.int32)
    pad = e_pad - n_edges
    src_p = jnp.concatenate([src, jnp.zeros((pad,), jnp.int32)])
    dst_p = jnp.concatenate([dst, jnp.full((pad,), DUMMY, jnp.int32)])
    srcs = src_p.reshape(NW, nchunk, CHUNK)
    dsts = dst_p.reshape(NW, nchunk, CHUNK)
    dsts_hist = dst_p.reshape(NW, ept // L, L)

    x_pad = jnp.zeros((N_PAD, D), x.dtype).at[:N_NODES].set(x)
    b1r = b1.reshape(1, D)
    b2r = b2.reshape(1, D)

    hist = _sc_hist(dsts_hist)
    dis, g1 = _tc_scale_in(hist, x_pad, W1)
    p = _sc_agg(g1, srcs, dsts)
    g2 = _tc_mid(p, g1, dis, b1r, W2)
    q = _sc_agg(g2, srcs, dsts)
    out = _tc_out(q, g2, dis, b2r)
    return out[:N_NODES]


# speedup vs baseline: 10.2532x; 1.1448x over previous
import functools

import jax
import jax.numpy as jnp
from jax import lax
from jax.experimental import pallas as pl
from jax.experimental.pallas import tpu as pltpu
from jax.experimental.pallas import tpu_sc as plsc

N_NODES = 10000
D = 128
N_PAD = 10240
DUMMY = N_NODES
NC = 2
NS = 16
L = 16
NW = NC * NS
CHUNK = 64
NBUF = 4
NSTAGE = 4
ROW_BLK = 1024

_SC_PARAMS = pltpu.CompilerParams(needs_layout_passes=False)


def _sc_hist(dsts):
    ept_l = dsts.shape[1]
    mesh = plsc.VectorSubcoreMesh(core_axis_name="c", subcore_axis_name="s")

    @functools.partial(
        pl.kernel, mesh=mesh, compiler_params=_SC_PARAMS,
        out_type=jax.ShapeDtypeStruct((NW, N_PAD), jnp.float32),
        scratch_types=[
            pltpu.VMEM((ept_l, L), jnp.int32),
            pltpu.VMEM((N_PAD,), jnp.float32),
        ],
    )
    def k(dst_hbm, out_hbm, idx_v, hist_v):
        cid = lax.axis_index("c")
        sid = lax.axis_index("s")
        wid = sid * NC + cid
        zeros16 = jnp.zeros((L,), jnp.float32)
        ones16 = jnp.ones((L,), jnp.float32)

        @pl.loop(0, N_PAD // L)
        def _(i):
            hist_v[pl.ds(i * L, L)] = zeros16

        pltpu.sync_copy(dst_hbm.at[wid], idx_v)

        @pl.loop(0, ept_l)
        def _(j):
            plsc.addupdate_scatter(hist_v, [idx_v[j]], ones16)

        pltpu.sync_copy(hist_v, out_hbm.at[wid])

    return k(dsts)


def _sc_agg(g, srcs, dsts):
    nchunk = srcs.shape[1]
    stage = nchunk // NSTAGE
    rows_per_tile = N_PAD // NS
    mesh = plsc.VectorSubcoreMesh(core_axis_name="c", subcore_axis_name="s")

    @functools.partial(
        pl.kernel, mesh=mesh,
        out_type=jax.ShapeDtypeStruct((NC, N_PAD, D), jnp.float32),
        scratch_types=[
            pltpu.VMEM((stage, CHUNK), jnp.int32),
            pltpu.VMEM((stage, CHUNK), jnp.int32),
            pltpu.VMEM((NBUF, CHUNK, D), jnp.float32),
            pltpu.VMEM_SHARED((N_PAD, D), jnp.float32),
        ] + [pltpu.SemaphoreType.DMA] * NBUF,
    )
    def k(g_hbm, src_hbm, dst_hbm, out_hbm,
          src_v, dst_v, bufs, acc, *sems):
        cid = lax.axis_index("c")
        sid = lax.axis_index("s")
        wid = sid * NC + cid
        zeros16 = jnp.zeros((L,), jnp.float32)

        @pl.loop(0, CHUNK)
        def _(r):
            @pl.loop(0, D // L)
            def _(c):
                bufs[0, r, pl.ds(c * L, L)] = zeros16

        @pl.loop(0, rows_per_tile // CHUNK)
        def _(t):
            pltpu.sync_copy(
                bufs.at[0],
                acc.at[pl.ds(sid * rows_per_tile + t * CHUNK, CHUNK)])

        plsc.subcore_barrier()

        @pl.loop(0, NSTAGE)
        def _(st):
            pltpu.sync_copy(src_hbm.at[wid].at[pl.ds(st * stage, stage)],
                            src_v)
            pltpu.sync_copy(dst_hbm.at[wid].at[pl.ds(st * stage, stage)],
                            dst_v)
            for b in range(NBUF):
                pltpu.async_copy(g_hbm.at[src_v.at[b]], bufs.at[b], sems[b])

            @pl.loop(0, stage, step=NBUF)
            def _(j):
                for b in range(NBUF):
                    pltpu.make_async_copy(g_hbm.at[src_v.at[j + b]],
                                          bufs.at[b], sems[b]).wait()
                    pltpu.sync_copy(bufs.at[b], acc.at[dst_v.at[j + b]],
                                    add=True)

                    @pl.when(j + b + NBUF < stage)
                    def _():
                        pltpu.async_copy(g_hbm.at[src_v.at[j + b + NBUF]],
                                         bufs.at[b], sems[b])

        plsc.subcore_barrier()
        pltpu.sync_copy(
            acc.at[pl.ds(sid * rows_per_tile, rows_per_tile)],
            out_hbm.at[cid].at[pl.ds(sid * rows_per_tile, rows_per_tile)])

    return k(g, srcs, dsts)


def _tc_scale_in(hist, x, w1):
    def body(hist_b, x_b, w1_b, dis_b, g1_b):
        deg = jnp.sum(hist_b[...], axis=0) + 1.0
        dis = lax.rsqrt(deg)
        h = jnp.dot(x_b[...], w1_b[...],
                    preferred_element_type=jnp.float32,
                    precision=lax.Precision.HIGHEST)
        dis_b[...] = dis[:, None]
        g1_b[...] = h * dis[:, None]

    grid = (N_PAD // ROW_BLK,)
    return pl.pallas_call(
        body,
        grid=grid,
        in_specs=[
            pl.BlockSpec((NW, ROW_BLK), lambda b: (0, b)),
            pl.BlockSpec((ROW_BLK, D), lambda b: (b, 0)),
            pl.BlockSpec((D, D), lambda b: (0, 0)),
        ],
        out_specs=[
            pl.BlockSpec((ROW_BLK, 1), lambda b: (b, 0)),
            pl.BlockSpec((ROW_BLK, D), lambda b: (b, 0)),
        ],
        out_shape=[
            jax.ShapeDtypeStruct((N_PAD, 1), jnp.float32),
            jax.ShapeDtypeStruct((N_PAD, D), jnp.float32),
        ],
    )(hist, x, w1)


def _tc_mid(p, g1, dis, b1, w2):
    def body(p_b, g1_b, dis_b, b1_b, w2_b, g2_b):
        dis = dis_b[...]
        a = (p_b[0] + p_b[1] + g1_b[...]) * dis + b1_b[...]
        a = jnp.maximum(a, 0.0)
        h2 = jnp.dot(a, w2_b[...],
                     preferred_element_type=jnp.float32,
                     precision=lax.Precision.HIGHEST)
        g2_b[...] = h2 * dis

    grid = (N_PAD // ROW_BLK,)
    return pl.pallas_call(
        body,
        grid=grid,
        in_specs=[
            pl.BlockSpec((NC, ROW_BLK, D), lambda b: (0, b, 0)),
            pl.BlockSpec((ROW_BLK, D), lambda b: (b, 0)),
            pl.BlockSpec((ROW_BLK, 1), lambda b: (b, 0)),
            pl.BlockSpec((1, D), lambda b: (0, 0)),
            pl.BlockSpec((D, D), lambda b: (0, 0)),
        ],
        out_specs=pl.BlockSpec((ROW_BLK, D), lambda b: (b, 0)),
        out_shape=jax.ShapeDtypeStruct((N_PAD, D), jnp.float32),
    )(p, g1, dis, b1, w2)


def _tc_out(q, g2, dis, b2):
    def body(q_b, g2_b, dis_b, b2_b, o_b):
        o_b[...] = (q_b[0] + q_b[1] + g2_b[...]) * dis_b[...] + b2_b[...]

    grid = (N_PAD // ROW_BLK,)
    return pl.pallas_call(
        body,
        grid=grid,
        in_specs=[
            pl.BlockSpec((NC, ROW_BLK, D), lambda b: (0, b, 0)),
            pl.BlockSpec((ROW_BLK, D), lambda b: (b, 0)),
            pl.BlockSpec((ROW_BLK, 1), lambda b: (b, 0)),
            pl.BlockSpec((1, D), lambda b: (0, 0)),
        ],
        out_specs=pl.BlockSpec((ROW_BLK, D), lambda b: (b, 0)),
        out_shape=jax.ShapeDtypeStruct((N_PAD, D), jnp.float32),
    )(q, g2, dis, b2)


def kernel(x, edge_index, W1, b1, W2, b2):
    n_edges = edge_index.shape[1]
    ept = -(-n_edges // NW)
    q = NSTAGE * 8 * CHUNK
    ept = -(-ept // q) * q
    e_pad = ept * NW
    nchunk = ept // CHUNK

    src = edge_index[0].astype(jnp.int32)
    dst = edge_index[1].astype(jnp.int32)
    pad = e_pad - n_edges
    src_p = jnp.concatenate([src, jnp.zeros((pad,), jnp.int32)])
    dst_p = jnp.concatenate([dst, jnp.full((pad,), DUMMY, jnp.int32)])
    srcs = src_p.reshape(NW, nchunk, CHUNK)
    dsts = dst_p.reshape(NW, nchunk, CHUNK)
    dsts_hist = dst_p.reshape(NW, ept // L, L)

    x_pad = jnp.zeros((N_PAD, D), x.dtype).at[:N_NODES].set(x)
    b1r = b1.reshape(1, D)
    b2r = b2.reshape(1, D)

    hist = _sc_hist(dsts_hist)
    dis, g1 = _tc_scale_in(hist, x_pad, W1)
    p = _sc_agg(g1, srcs, dsts)
    g2 = _tc_mid(p, g1, dis, b1r, W2)
    q = _sc_agg(g2, srcs, dsts)
    out = _tc_out(q, g2, dis, b2r)
    return out[:N_NODES]


# speedup vs baseline: 11.3612x; 1.1081x over previous
import functools

import jax
import jax.numpy as jnp
from jax import lax
from jax.experimental import pallas as pl
from jax.experimental.pallas import tpu as pltpu
from jax.experimental.pallas import tpu_sc as plsc

N_NODES = 10000
D = 128
N_PAD = 10240
DUMMY = N_NODES
NC = 2
NS = 16
L = 16
NW = NC * NS
CHUNK = 32
NBUF = 8
NSTAGE = 8
ROW_BLK = 1024

_SC_PARAMS = pltpu.CompilerParams(needs_layout_passes=False)


def _sc_hist(dsts):
    ept_l = dsts.shape[1]
    mesh = plsc.VectorSubcoreMesh(core_axis_name="c", subcore_axis_name="s")

    @functools.partial(
        pl.kernel, mesh=mesh, compiler_params=_SC_PARAMS,
        out_type=jax.ShapeDtypeStruct((NW, N_PAD), jnp.float32),
        scratch_types=[
            pltpu.VMEM((ept_l, L), jnp.int32),
            pltpu.VMEM((N_PAD,), jnp.float32),
        ],
    )
    def k(dst_hbm, out_hbm, idx_v, hist_v):
        cid = lax.axis_index("c")
        sid = lax.axis_index("s")
        wid = sid * NC + cid
        zeros16 = jnp.zeros((L,), jnp.float32)
        ones16 = jnp.ones((L,), jnp.float32)

        @pl.loop(0, N_PAD // L)
        def _(i):
            hist_v[pl.ds(i * L, L)] = zeros16

        pltpu.sync_copy(dst_hbm.at[wid], idx_v)

        @pl.loop(0, ept_l)
        def _(j):
            plsc.addupdate_scatter(hist_v, [idx_v[j]], ones16)

        pltpu.sync_copy(hist_v, out_hbm.at[wid])

    return k(dsts)


def _sc_agg(g, srcs, dsts):
    nchunk = srcs.shape[1]
    stage = nchunk // NSTAGE
    rows_per_tile = N_PAD // NS
    dt = g.dtype
    vw = 32 if dt == jnp.bfloat16 else L
    mesh = plsc.VectorSubcoreMesh(core_axis_name="c", subcore_axis_name="s")

    @functools.partial(
        pl.kernel, mesh=mesh,
        out_type=jax.ShapeDtypeStruct((NC, N_PAD, D), dt),
        scratch_types=[
            pltpu.VMEM((stage, CHUNK), jnp.int32),
            pltpu.VMEM((stage, CHUNK), jnp.int32),
            pltpu.VMEM((NBUF, CHUNK, D), dt),
            pltpu.VMEM_SHARED((N_PAD, D), dt),
        ] + [pltpu.SemaphoreType.DMA] * NBUF,
    )
    def k(g_hbm, src_hbm, dst_hbm, out_hbm,
          src_v, dst_v, bufs, acc, *sems):
        cid = lax.axis_index("c")
        sid = lax.axis_index("s")
        wid = sid * NC + cid
        if dt == jnp.bfloat16:
            zblk = jnp.zeros((2, L), dt)

            @pl.loop(0, CHUNK, step=2)
            def _(r):
                @pl.loop(0, D // L)
                def _(c):
                    bufs[0, pl.ds(r, 2), pl.ds(c * L, L)] = zblk
        else:
            zvec = jnp.zeros((vw,), dt)

            @pl.loop(0, CHUNK)
            def _(r):
                @pl.loop(0, D // vw)
                def _(c):
                    bufs[0, r, pl.ds(c * vw, vw)] = zvec

        @pl.loop(0, rows_per_tile // CHUNK)
        def _(t):
            pltpu.sync_copy(
                bufs.at[0],
                acc.at[pl.ds(sid * rows_per_tile + t * CHUNK, CHUNK)])

        plsc.subcore_barrier()

        @pl.loop(0, NSTAGE)
        def _(st):
            pltpu.sync_copy(src_hbm.at[wid].at[pl.ds(st * stage, stage)],
                            src_v)
            pltpu.sync_copy(dst_hbm.at[wid].at[pl.ds(st * stage, stage)],
                            dst_v)
            for b in range(NBUF):
                pltpu.async_copy(g_hbm.at[src_v.at[b]], bufs.at[b], sems[b])

            @pl.loop(0, stage, step=NBUF)
            def _(j):
                for b in range(NBUF):
                    pltpu.make_async_copy(g_hbm.at[src_v.at[j + b]],
                                          bufs.at[b], sems[b]).wait()
                    pltpu.sync_copy(bufs.at[b], acc.at[dst_v.at[j + b]],
                                    add=True)

                    @pl.when(j + b + NBUF < stage)
                    def _():
                        pltpu.async_copy(g_hbm.at[src_v.at[j + b + NBUF]],
                                         bufs.at[b], sems[b])

        plsc.subcore_barrier()
        pltpu.sync_copy(
            acc.at[pl.ds(sid * rows_per_tile, rows_per_tile)],
            out_hbm.at[cid].at[pl.ds(sid * rows_per_tile, rows_per_tile)])

    return k(g, srcs, dsts)


def _tc_scale_in(hist, x, w1):
    def body(hist_b, x_b, w1_b, dis_b, g1_b):
        deg = jnp.sum(hist_b[...], axis=0) + 1.0
        dis = lax.rsqrt(deg)
        h = jnp.dot(x_b[...], w1_b[...],
                    preferred_element_type=jnp.float32,
                    precision=lax.Precision.HIGHEST)
        dis_b[...] = dis[:, None]
        g1_b[...] = h * dis[:, None]

    grid = (N_PAD // ROW_BLK,)
    return pl.pallas_call(
        body,
        grid=grid,
        in_specs=[
            pl.BlockSpec((NW, ROW_BLK), lambda b: (0, b)),
            pl.BlockSpec((ROW_BLK, D), lambda b: (b, 0)),
            pl.BlockSpec((D, D), lambda b: (0, 0)),
        ],
        out_specs=[
            pl.BlockSpec((ROW_BLK, 1), lambda b: (b, 0)),
            pl.BlockSpec((ROW_BLK, D), lambda b: (b, 0)),
        ],
        out_shape=[
            jax.ShapeDtypeStruct((N_PAD, 1), jnp.float32),
            jax.ShapeDtypeStruct((N_PAD, D), jnp.float32),
        ],
    )(hist, x, w1)


def _tc_mid(p, g1, dis, b1, w2):
    def body(p_b, g1_b, dis_b, b1_b, w2_b, g2_b):
        dis = dis_b[...]
        agg = p_b[0].astype(jnp.float32) + p_b[1].astype(jnp.float32)
        a = (agg + g1_b[...]) * dis + b1_b[...]
        a = jnp.maximum(a, 0.0)
        h2 = jnp.dot(a, w2_b[...],
                     preferred_element_type=jnp.float32,
                     precision=lax.Precision.HIGHEST)
        g2_b[...] = h2 * dis

    grid = (N_PAD // ROW_BLK,)
    return pl.pallas_call(
        body,
        grid=grid,
        in_specs=[
            pl.BlockSpec((NC, ROW_BLK, D), lambda b: (0, b, 0)),
            pl.BlockSpec((ROW_BLK, D), lambda b: (b, 0)),
            pl.BlockSpec((ROW_BLK, 1), lambda b: (b, 0)),
            pl.BlockSpec((1, D), lambda b: (0, 0)),
            pl.BlockSpec((D, D), lambda b: (0, 0)),
        ],
        out_specs=pl.BlockSpec((ROW_BLK, D), lambda b: (b, 0)),
        out_shape=jax.ShapeDtypeStruct((N_PAD, D), jnp.float32),
    )(p, g1, dis, b1, w2)


def _tc_out(q, g2, dis, b2):
    def body(q_b, g2_b, dis_b, b2_b, o_b):
        agg = q_b[0].astype(jnp.float32) + q_b[1].astype(jnp.float32)
        o_b[...] = (agg + g2_b[...]) * dis_b[...] + b2_b[...]

    grid = (N_PAD // ROW_BLK,)
    return pl.pallas_call(
        body,
        grid=grid,
        in_specs=[
            pl.BlockSpec((NC, ROW_BLK, D), lambda b: (0, b, 0)),
            pl.BlockSpec((ROW_BLK, D), lambda b: (b, 0)),
            pl.BlockSpec((ROW_BLK, 1), lambda b: (b, 0)),
            pl.BlockSpec((1, D), lambda b: (0, 0)),
        ],
        out_specs=pl.BlockSpec((ROW_BLK, D), lambda b: (b, 0)),
        out_shape=jax.ShapeDtypeStruct((N_PAD, D), jnp.float32),
    )(q, g2, dis, b2)


def kernel(x, edge_index, W1, b1, W2, b2):
    n_edges = edge_index.shape[1]
    ept = -(-n_edges // NW)
    q = NSTAGE * 8 * CHUNK
    ept = -(-ept // q) * q
    e_pad = ept * NW
    nchunk = ept // CHUNK

    src = edge_index[0].astype(jnp.int32)
    dst = edge_index[1].astype(jnp.int32)
    pad = e_pad - n_edges
    src_p = jnp.concatenate([src, jnp.zeros((pad,), jnp.int32)])
    dst_p = jnp.concatenate([dst, jnp.full((pad,), DUMMY, jnp.int32)])
    srcs = src_p.reshape(NW, nchunk, CHUNK)
    dsts = dst_p.reshape(NW, nchunk, CHUNK)
    dsts_hist = dst_p.reshape(NW, ept // L, L)

    x_pad = jnp.zeros((N_PAD, D), x.dtype).at[:N_NODES].set(x)
    b1r = b1.reshape(1, D)
    b2r = b2.reshape(1, D)

    hist = _sc_hist(dsts_hist)
    dis, g1 = _tc_scale_in(hist, x_pad, W1)
    p = _sc_agg(g1, srcs, dsts)
    g2 = _tc_mid(p, g1, dis, b1r, W2)
    q = _sc_agg(g2, srcs, dsts)
    out = _tc_out(q, g2, dis, b2r)
    return out[:N_NODES]


# speedup vs baseline: 11.3876x; 1.0023x over previous
import functools

import jax
import jax.numpy as jnp
from jax import lax
from jax.experimental import pallas as pl
from jax.experimental.pallas import tpu as pltpu
from jax.experimental.pallas import tpu_sc as plsc

N_NODES = 10000
D = 128
N_PAD = 10240
DUMMY = N_NODES
NC = 2
NS = 16
L = 16
NW = NC * NS
CHUNK = 32
NBUF = 8
NSTAGE = 8
SDEPTH = 2
ROW_BLK = 1024

_SC_PARAMS = pltpu.CompilerParams(needs_layout_passes=False)


def _sc_hist(dsts):
    ept_l = dsts.shape[1]
    mesh = plsc.VectorSubcoreMesh(core_axis_name="c", subcore_axis_name="s")

    @functools.partial(
        pl.kernel, mesh=mesh, compiler_params=_SC_PARAMS,
        out_type=jax.ShapeDtypeStruct((NW, N_PAD), jnp.float32),
        scratch_types=[
            pltpu.VMEM((ept_l, L), jnp.int32),
            pltpu.VMEM((N_PAD,), jnp.float32),
        ],
    )
    def k(dst_hbm, out_hbm, idx_v, hist_v):
        cid = lax.axis_index("c")
        sid = lax.axis_index("s")
        wid = sid * NC + cid
        zeros16 = jnp.zeros((L,), jnp.float32)
        ones16 = jnp.ones((L,), jnp.float32)

        @pl.loop(0, N_PAD // L)
        def _(i):
            hist_v[pl.ds(i * L, L)] = zeros16

        pltpu.sync_copy(dst_hbm.at[wid], idx_v)

        @pl.loop(0, ept_l)
        def _(j):
            plsc.addupdate_scatter(hist_v, [idx_v[j]], ones16)

        pltpu.sync_copy(hist_v, out_hbm.at[wid])

    return k(dsts)


def _sc_agg(g, srcs, dsts):
    nchunk = srcs.shape[1]
    stage = nchunk // NSTAGE
    rows_per_tile = N_PAD // NS
    dt = g.dtype
    vw = 32 if dt == jnp.bfloat16 else L
    mesh = plsc.VectorSubcoreMesh(core_axis_name="c", subcore_axis_name="s")

    @functools.partial(
        pl.kernel, mesh=mesh,
        out_type=jax.ShapeDtypeStruct((NC, N_PAD, D), dt),
        scratch_types=[
            pltpu.VMEM((stage, CHUNK), jnp.int32),
            pltpu.VMEM((stage, CHUNK), jnp.int32),
            pltpu.VMEM((NBUF, CHUNK, D), dt),
            pltpu.VMEM_SHARED((N_PAD, D), dt),
        ] + [pltpu.SemaphoreType.DMA] * (2 * NBUF),
    )
    def k(g_hbm, src_hbm, dst_hbm, out_hbm,
          src_v, dst_v, bufs, acc, *sems):
        sem_g, sem_s = sems[:NBUF], sems[NBUF:]
        cid = lax.axis_index("c")
        sid = lax.axis_index("s")
        wid = sid * NC + cid
        if dt == jnp.bfloat16:
            zblk = jnp.zeros((2, L), dt)

            @pl.loop(0, CHUNK, step=2)
            def _(r):
                @pl.loop(0, D // L)
                def _(c):
                    bufs[0, pl.ds(r, 2), pl.ds(c * L, L)] = zblk
        else:
            zvec = jnp.zeros((vw,), dt)

            @pl.loop(0, CHUNK)
            def _(r):
                @pl.loop(0, D // vw)
                def _(c):
                    bufs[0, r, pl.ds(c * vw, vw)] = zvec

        @pl.loop(0, rows_per_tile // CHUNK)
        def _(t):
            pltpu.sync_copy(
                bufs.at[0],
                acc.at[pl.ds(sid * rows_per_tile + t * CHUNK, CHUNK)])

        plsc.subcore_barrier()

        @pl.loop(0, NSTAGE)
        def _(st):
            pltpu.sync_copy(src_hbm.at[wid].at[pl.ds(st * stage, stage)],
                            src_v)
            pltpu.sync_copy(dst_hbm.at[wid].at[pl.ds(st * stage, stage)],
                            dst_v)
            for b in range(NBUF):
                pltpu.async_copy(g_hbm.at[src_v.at[b]], bufs.at[b],
                                 sem_g[b])

            @pl.loop(0, stage, step=NBUF)
            def _(j):
                for b in range(NBUF):
                    slot_r = (b - SDEPTH) % NBUF

                    @pl.when((j + b >= SDEPTH)
                             & (j + b - SDEPTH + NBUF < stage))
                    def _():
                        pltpu.make_async_copy(
                            bufs.at[slot_r],
                            acc.at[dst_v.at[j + b - SDEPTH]],
                            sem_s[slot_r]).wait()
                        pltpu.async_copy(
                            g_hbm.at[src_v.at[j + b - SDEPTH + NBUF]],
                            bufs.at[slot_r], sem_g[slot_r])

                    pltpu.make_async_copy(g_hbm.at[src_v.at[j + b]],
                                          bufs.at[b], sem_g[b]).wait()
                    pltpu.async_copy(bufs.at[b], acc.at[dst_v.at[j + b]],
                                     sem_s[b], add=True)

            for b in range(NBUF):
                pltpu.make_async_copy(bufs.at[b],
                                      acc.at[dst_v.at[stage - NBUF + b]],
                                      sem_s[b]).wait()

        plsc.subcore_barrier()
        pltpu.sync_copy(
            acc.at[pl.ds(sid * rows_per_tile, rows_per_tile)],
            out_hbm.at[cid].at[pl.ds(sid * rows_per_tile, rows_per_tile)])

    return k(g, srcs, dsts)


def _tc_scale_in(hist, x, w1):
    def body(hist_b, x_b, w1_b, dis_b, g1_b):
        deg = jnp.sum(hist_b[...], axis=0) + 1.0
        dis = lax.rsqrt(deg)
        h = jnp.dot(x_b[...], w1_b[...],
                    preferred_element_type=jnp.float32,
                    precision=lax.Precision.HIGHEST)
        dis_b[...] = dis[:, None]
        g1_b[...] = h * dis[:, None]

    grid = (N_PAD // ROW_BLK,)
    return pl.pallas_call(
        body,
        grid=grid,
        in_specs=[
            pl.BlockSpec((NW, ROW_BLK), lambda b: (0, b)),
            pl.BlockSpec((ROW_BLK, D), lambda b: (b, 0)),
            pl.BlockSpec((D, D), lambda b: (0, 0)),
        ],
        out_specs=[
            pl.BlockSpec((ROW_BLK, 1), lambda b: (b, 0)),
            pl.BlockSpec((ROW_BLK, D), lambda b: (b, 0)),
        ],
        out_shape=[
            jax.ShapeDtypeStruct((N_PAD, 1), jnp.float32),
            jax.ShapeDtypeStruct((N_PAD, D), jnp.float32),
        ],
    )(hist, x, w1)


def _tc_mid(p, g1, dis, b1, w2):
    def body(p_b, g1_b, dis_b, b1_b, w2_b, g2_b):
        dis = dis_b[...]
        agg = p_b[0].astype(jnp.float32) + p_b[1].astype(jnp.float32)
        a = (agg + g1_b[...]) * dis + b1_b[...]
        a = jnp.maximum(a, 0.0)
        h2 = jnp.dot(a, w2_b[...],
                     preferred_element_type=jnp.float32,
                     precision=lax.Precision.HIGHEST)
        g2_b[...] = h2 * dis

    grid = (N_PAD // ROW_BLK,)
    return pl.pallas_call(
        body,
        grid=grid,
        in_specs=[
            pl.BlockSpec((NC, ROW_BLK, D), lambda b: (0, b, 0)),
            pl.BlockSpec((ROW_BLK, D), lambda b: (b, 0)),
            pl.BlockSpec((ROW_BLK, 1), lambda b: (b, 0)),
            pl.BlockSpec((1, D), lambda b: (0, 0)),
            pl.BlockSpec((D, D), lambda b: (0, 0)),
        ],
        out_specs=pl.BlockSpec((ROW_BLK, D), lambda b: (b, 0)),
        out_shape=jax.ShapeDtypeStruct((N_PAD, D), jnp.float32),
    )(p, g1, dis, b1, w2)


def _tc_out(q, g2, dis, b2):
    def body(q_b, g2_b, dis_b, b2_b, o_b):
        agg = q_b[0].astype(jnp.float32) + q_b[1].astype(jnp.float32)
        o_b[...] = (agg + g2_b[...]) * dis_b[...] + b2_b[...]

    grid = (N_PAD // ROW_BLK,)
    return pl.pallas_call(
        body,
        grid=grid,
        in_specs=[
            pl.BlockSpec((NC, ROW_BLK, D), lambda b: (0, b, 0)),
            pl.BlockSpec((ROW_BLK, D), lambda b: (b, 0)),
            pl.BlockSpec((ROW_BLK, 1), lambda b: (b, 0)),
            pl.BlockSpec((1, D), lambda b: (0, 0)),
        ],
        out_specs=pl.BlockSpec((ROW_BLK, D), lambda b: (b, 0)),
        out_shape=jax.ShapeDtypeStruct((N_PAD, D), jnp.float32),
    )(q, g2, dis, b2)


def kernel(x, edge_index, W1, b1, W2, b2):
    n_edges = edge_index.shape[1]
    ept = -(-n_edges // NW)
    q = NSTAGE * 8 * CHUNK
    ept = -(-ept // q) * q
    e_pad = ept * NW
    nchunk = ept // CHUNK

    src = edge_index[0].astype(jnp.int32)
    dst = edge_index[1].astype(jnp.int32)
    pad = e_pad - n_edges
    src_p = jnp.concatenate([src, jnp.zeros((pad,), jnp.int32)])
    dst_p = jnp.concatenate([dst, jnp.full((pad,), DUMMY, jnp.int32)])
    srcs = src_p.reshape(NW, nchunk, CHUNK)
    dsts = dst_p.reshape(NW, nchunk, CHUNK)
    dsts_hist = dst_p.reshape(NW, ept // L, L)

    x_pad = jnp.zeros((N_PAD, D), x.dtype).at[:N_NODES].set(x)
    b1r = b1.reshape(1, D)
    b2r = b2.reshape(1, D)

    hist = _sc_hist(dsts_hist)
    dis, g1 = _tc_scale_in(hist, x_pad, W1)
    p = _sc_agg(g1, srcs, dsts)
    g2 = _tc_mid(p, g1, dis, b1r, W2)
    q = _sc_agg(g2, srcs, dsts)
    out = _tc_out(q, g2, dis, b2r)
    return out[:N_NODES]
